# Initial kernel scaffold; baseline (speedup 1.0000x reference)
import jax
import jax.numpy as jnp
from jax.experimental import pallas as pl


def kernel(input_pc, params):
    raise NotImplementedError("write your pallas kernel here")


# speedup vs baseline: 5.1189x; 5.1189x over previous
import functools

import jax
import jax.numpy as jnp
import numpy as np
from jax.experimental import pallas as pl

_INV = np.float32(1.0 / np.sqrt(1.0 + 1e-5))


def _dot(a, b):
    return jnp.dot(a.astype(jnp.bfloat16), b.astype(jnp.bfloat16),
                   preferred_element_type=jnp.float32)


def _mlp(x, layers):
    for (W, b, g, be) in layers:
        x = _dot(x, W) + b
        x = jax.nn.relu(g * (x * _INV) + be)
    return x


def _fps_kernel(xT_ref, out_ref, *, npoint):
    xT = xT_ref[...]
    B = xT.shape[0]
    N = xT.shape[2]
    x0 = xT[:, 0, :]
    x1 = xT[:, 1, :]
    x2 = xT[:, 2, :]
    iota = jax.lax.broadcasted_iota(jnp.int32, (B, N), 1)
    iota_np = jax.lax.broadcasted_iota(jnp.int32, (B, npoint), 1)

    def body(i, carry):
        distance, farthest, a0, a1, a2 = carry
        onehot = (iota == farthest[:, None]).astype(jnp.float32)
        c0 = jnp.sum(onehot * x0, -1)
        c1 = jnp.sum(onehot * x1, -1)
        c2 = jnp.sum(onehot * x2, -1)
        hit = iota_np == i
        a0 = jnp.where(hit, c0[:, None], a0)
        a1 = jnp.where(hit, c1[:, None], a1)
        a2 = jnp.where(hit, c2[:, None], a2)
        d = ((x0 - c0[:, None]) ** 2 + (x1 - c1[:, None]) ** 2
             + (x2 - c2[:, None]) ** 2)
        distance = jnp.minimum(distance, d)
        maxv = jnp.max(distance, -1)
        farthest = jnp.min(jnp.where(distance == maxv[:, None], iota, N), -1)
        return distance, farthest, a0, a1, a2

    zer = jnp.zeros((B, npoint), jnp.float32)
    init = (jnp.full((B, N), 1e10, jnp.float32),
            jnp.zeros((B,), jnp.int32), zer, zer, zer)
    _, _, a0, a1, a2 = jax.lax.fori_loop(0, npoint, body, init)
    out_ref[:, 0, :] = a0
    out_ref[:, 1, :] = a1
    out_ref[:, 2, :] = a2


def _fps_call(xT, npoint):
    B = xT.shape[0]
    return pl.pallas_call(
        functools.partial(_fps_kernel, npoint=npoint),
        out_shape=jax.ShapeDtypeStruct((B, 3, npoint), jnp.float32),
    )(xT)


def _sa_kernel(nx_ref, s2_ref, xT_ref, d2_ref, P_ref, *rest, K, radius2, C,
               nlayers):
    out_ref = rest[-1]
    wrefs = rest[:-1]
    layers = [tuple(wrefs[4 * i + j][...] for j in range(4))
              for i in range(nlayers)]
    nx = nx_ref[0]
    s2 = s2_ref[0]
    xT = xT_ref[0]
    d2 = d2_ref[0]
    P = P_ref[0]
    Sb = nx.shape[0]
    N = xT.shape[1]
    fN = np.float32(N)

    sqr = s2 + d2 - 2.0 * _dot(nx, xT)
    iota = jax.lax.broadcasted_iota(jnp.int32, (Sb, N), 1).astype(jnp.float32)
    vals = jnp.where(sqr > radius2, fN, iota)
    idx0 = jnp.min(vals, -1)
    offs = jnp.concatenate(
        [jnp.zeros((Sb, C), jnp.float32), nx], axis=-1)

    Cout = out_ref.shape[-1]

    def body(_, carry):
        out, cur = carry
        mk = jnp.min(cur, -1)
        sel = jnp.where(mk >= fN, idx0, mk)
        onehot = (iota == sel[:, None]).astype(jnp.float32)
        g = jnp.dot(onehot, P, preferred_element_type=jnp.float32, precision=jax.lax.Precision.HIGHEST)
        h = _mlp(g - offs, layers)
        out = jnp.maximum(out, h)
        cur = jnp.where(cur == mk[:, None], fN, cur)
        return out, cur

    init = (jnp.full((Sb, Cout), -jnp.inf, jnp.float32), vals)
    out, _ = jax.lax.fori_loop(0, K, body, init)
    out_ref[0] = out


def _sa_branch(new_xyz, s2, xT, d2, P, layers, K, radius, s_blk):
    B, S, _ = new_xyz.shape
    N = xT.shape[2]
    C = P.shape[2] - 3
    wlist = []
    for p_ in layers:
        wlist += [p_['W'], p_['b'].reshape(1, -1),
                  p_['g'].reshape(1, -1), p_['be'].reshape(1, -1)]
    Cout = layers[-1]['W'].shape[1]
    grid = (B, S // s_blk)
    in_specs = [
        pl.BlockSpec((1, s_blk, 3), lambda b, s: (b, s, 0)),
        pl.BlockSpec((1, s_blk, 1), lambda b, s: (b, s, 0)),
        pl.BlockSpec((1, 3, N), lambda b, s: (b, 0, 0)),
        pl.BlockSpec((1, 1, N), lambda b, s: (b, 0, 0)),
        pl.BlockSpec((1, N, C + 3), lambda b, s: (b, 0, 0)),
    ] + [pl.BlockSpec(w.shape, lambda b, s: (0, 0)) for w in wlist]
    return pl.pallas_call(
        functools.partial(_sa_kernel, K=K, radius2=np.float32(radius * radius),
                          C=C, nlayers=len(layers)),
        grid=grid,
        in_specs=in_specs,
        out_specs=pl.BlockSpec((1, s_blk, Cout), lambda b, s: (b, s, 0)),
        out_shape=jax.ShapeDtypeStruct((B, S, Cout), jnp.float32),
    )(new_xyz, s2, xT, d2, P, *wlist)


def _fp_kernel(*refs, nlayers, has_p1, has_head):
    it = iter(refs)
    x1_ref = next(it)
    s1_ref = next(it)
    x2T_ref = next(it)
    s2_ref = next(it)
    p2_ref = next(it)
    p1_ref = next(it) if has_p1 else None
    rest = list(it)
    out_ref = rest[-1]
    wrefs = rest[:-1]
    layers = [tuple(wrefs[4 * i + j][...] for j in range(4))
              for i in range(nlayers)]
    pos = 4 * nlayers
    if has_head:
        hW1, hb1, hg1, hbe1, hW2, hb2 = [w[...] for w in wrefs[pos:pos + 6]]

    x1 = x1_ref[0]
    s1 = s1_ref[0]
    x2T = x2T_ref[0]
    s2 = s2_ref[0]
    p2 = p2_ref[0]
    Sb = x1.shape[0]
    S2 = x2T.shape[1]
    fS2 = np.float32(S2)

    d = s1 + s2 - 2.0 * _dot(x1, x2T)
    iota = jax.lax.broadcasted_iota(jnp.int32, (Sb, S2), 1).astype(jnp.float32)

    mks, gs = [], []
    cur = d
    for _ in range(3):
        mk = jnp.min(cur, -1)
        selpos = jnp.min(jnp.where(cur == mk[:, None], iota, fS2), -1)
        hit = iota == selpos[:, None]
        onehot = hit.astype(jnp.float32)
        gs.append(jnp.dot(onehot, p2, preferred_element_type=jnp.float32, precision=jax.lax.Precision.HIGHEST))
        cur = jnp.where(hit, jnp.inf, cur)
        mks.append(mk)

    recips = [1.0 / (mk + np.float32(1e-8)) for mk in mks]
    wsum = recips[0] + recips[1] + recips[2]
    interp = sum((r / wsum)[:, None] * g for r, g in zip(recips, gs))

    if has_p1:
        x = jnp.concatenate([p1_ref[0], interp], axis=-1)
    else:
        x = interp
    h = _mlp(x, layers)
    if has_head:
        h = _dot(h, hW1) + hb1
        h = jax.nn.relu(hg1 * (h * _INV) + hbe1)
        h = _dot(h, hW2) + hb2
    out_ref[0] = h


def _fp_call(xyz1, xyz2, points1, points2, layers, s_blk, head=None):
    B, S1, _ = xyz1.shape
    S2 = xyz2.shape[1]
    C2 = points2.shape[2]
    x2T = jnp.transpose(xyz2, (0, 2, 1))
    wlist = []
    for p_ in layers:
        wlist += [p_['W'], p_['b'].reshape(1, -1),
                  p_['g'].reshape(1, -1), p_['be'].reshape(1, -1)]
    if head is not None:
        wlist += [head['W1'], head['b1'].reshape(1, -1),
                  head['g1'].reshape(1, -1), head['be1'].reshape(1, -1),
                  head['W2'], head['b2'].reshape(1, -1)]
        Cout = head['W2'].shape[1]
    else:
        Cout = layers[-1]['W'].shape[1]
    s1 = jnp.sum(xyz1 ** 2, -1)[..., None]
    s2n = jnp.sum(xyz2 ** 2, -1)[:, None, :]
    grid = (B, S1 // s_blk)
    operands = [xyz1, s1, x2T, s2n, points2]
    in_specs = [
        pl.BlockSpec((1, s_blk, 3), lambda b, s: (b, s, 0)),
        pl.BlockSpec((1, s_blk, 1), lambda b, s: (b, s, 0)),
        pl.BlockSpec((1, 3, S2), lambda b, s: (b, 0, 0)),
        pl.BlockSpec((1, 1, S2), lambda b, s: (b, 0, 0)),
        pl.BlockSpec((1, S2, C2), lambda b, s: (b, 0, 0)),
    ]
    if points1 is not None:
        operands.append(points1)
        C1 = points1.shape[2]
        in_specs.append(pl.BlockSpec((1, s_blk, C1), lambda b, s: (b, s, 0)))
    operands += wlist
    in_specs += [pl.BlockSpec(w.shape, lambda b, s: (0, 0)) for w in wlist]
    return pl.pallas_call(
        functools.partial(_fp_kernel, nlayers=len(layers),
                          has_p1=points1 is not None, has_head=head is not None),
        grid=grid,
        in_specs=in_specs,
        out_specs=pl.BlockSpec((1, s_blk, Cout), lambda b, s: (b, s, 0)),
        out_shape=jax.ShapeDtypeStruct((B, S1, Cout), jnp.float32),
    )(*operands)


_SA = [
    (1024, [1.0, 3.0], [8, 32], 256),
    (512, [2.0, 4.0], [8, 32], 512),
    (256, [3.0, 6.0], [16, 32], 256),
    (128, [4.0, 8.0], [16, 32], 128),
]
_FP_BLK = [256, 512, 512, 512]


def kernel(input_pc, params):
    xyz0 = input_pc[:, :, :3]
    feat0 = input_pc[:, :, 3:]

    xs = [xyz0]
    ps = [feat0]
    x, p = xyz0, feat0
    for li, (npoint, radii, Ks, s_blk) in enumerate(_SA):
        xT = jnp.transpose(x, (0, 2, 1))
        new_xyz = jnp.transpose(_fps_call(xT, npoint), (0, 2, 1))
        P = jnp.concatenate([p, x], axis=-1)
        s2 = jnp.sum(new_xyz ** 2, -1)[..., None]
        d2 = jnp.sum(x ** 2, -1)[:, None, :]
        outs = [
            _sa_branch(new_xyz, s2, xT, d2, P, params['sa'][li][bi], Ks[bi],
                       radii[bi], s_blk)
            for bi in range(len(radii))
        ]
        x = new_xyz
        p = jnp.concatenate(outs, axis=-1)
        xs.append(x)
        ps.append(p)

    p3 = _fp_call(xs[3], xs[4], ps[3], ps[4], params['fp'][0], _FP_BLK[0])
    p2 = _fp_call(xs[2], xs[3], ps[2], p3, params['fp'][1], _FP_BLK[1])
    p1 = _fp_call(xs[1], xs[2], ps[1], p2, params['fp'][2], _FP_BLK[2])
    out = _fp_call(xs[0], xs[1], None, p1, params['fp'][3], _FP_BLK[3],
                   head=params['head'])
    return out
